# baseline (speedup 1.0000x reference)
import functools

import jax
import jax.numpy as jnp
from jax import lax
from jax.experimental import pallas as pl
from jax.experimental.pallas import tpu as pltpu
from jax.experimental.pallas import tpu_sc as plsc

_INFO = plsc.get_sparse_core_info()
_NC, _NS, _L = _INFO.num_cores, _INFO.num_subcores, _INFO.num_lanes
_NW = _NC * _NS

_CHUNK = 128
_NBUF = 5
_RUNROLL = 16
_WIDE = 128


@functools.lru_cache(maxsize=None)
def _make(V, D, B):
    assert B % (_NW * _CHUNK * _NBUF) == 0 and D % _L == 0
    b_per_w = B // _NW
    steps = b_per_w // _CHUNK
    rounds = steps // _NBUF
    mesh = plsc.VectorSubcoreMesh(core_axis_name="c", subcore_axis_name="s")

    scratch = (
        [pltpu.VMEM((steps, _CHUNK), jnp.int32)]
        + [pltpu.VMEM((_CHUNK, D), jnp.float32) for _ in range(2 * _NBUF)]
        + [pltpu.SemaphoreType.DMA for _ in range(2 * _NBUF)]
    )

    @functools.partial(
        pl.kernel,
        out_type=jax.ShapeDtypeStruct((B, _WIDE), jnp.float32),
        mesh=mesh,
        scratch_types=scratch,
        compiler_params=pltpu.CompilerParams(use_tc_tiling_on_sc=False),
    )
    def emb_kernel(idx_hbm, table_hbm, out_hbm, idx_v, *bufs_and_sems):
        in_bufs = bufs_and_sems[:_NBUF]
        out_bufs = bufs_and_sems[_NBUF:2 * _NBUF]
        g_sems = bufs_and_sems[2 * _NBUF:3 * _NBUF]
        w_sems = bufs_and_sems[3 * _NBUF:4 * _NBUF]

        wid = lax.axis_index("s") * _NC + lax.axis_index("c")
        row_base = wid * b_per_w

        pltpu.sync_copy(idx_hbm.at[pl.ds(wid * steps, steps)], idx_v)

        def fire_gather(b, c):
            pltpu.async_copy(table_hbm.at[idx_v.at[c]], in_bufs[b], g_sems[b])

        def wait_gather(b):
            pltpu.make_async_copy(
                table_hbm.at[pl.ds(0, _CHUNK)], in_bufs[b], g_sems[b]).wait()

        def out_slice(c):
            return out_hbm.at[pl.ds(row_base + c * _CHUNK, _CHUNK),
                              pl.ds(0, D)]

        def fire_wb(b, c):
            pltpu.async_copy(out_bufs[b], out_slice(c), w_sems[b])

        def wait_wb(b):
            pltpu.make_async_copy(out_bufs[b], out_slice(0), w_sems[b]).wait()

        def relu_chunk(b):
            src, dst = in_bufs[b], out_bufs[b]

            def body(i, carry):
                r0 = i * _RUNROLL
                for r in range(_RUNROLL):
                    for j in range(D // _L):
                        sl = pl.ds(j * _L, _L)
                        dst[r0 + r, sl] = jnp.maximum(src[r0 + r, sl], 0.0)
                return carry

            lax.fori_loop(0, _CHUNK // _RUNROLL, body, 0)

        for b in range(_NBUF):
            fire_gather(b, b)

        for b in range(_NBUF):
            wait_gather(b)
            relu_chunk(b)
            fire_wb(b, b)
            fire_gather(b, _NBUF + b)

        def round_body(p, carry):
            c0 = p * _NBUF
            for b in range(_NBUF):
                wait_gather(b)
                wait_wb(b)
                relu_chunk(b)
                fire_wb(b, c0 + b)
                fire_gather(b, c0 + _NBUF + b)
            return carry

        lax.fori_loop(1, rounds - 1, round_body, 0)

        c0 = (rounds - 1) * _NBUF
        for b in range(_NBUF):
            wait_gather(b)
            wait_wb(b)
            relu_chunk(b)
            fire_wb(b, c0 + b)
        for b in range(_NBUF):
            wait_wb(b)

    return emb_kernel


def kernel(x, table):
    B = x.shape[0] * x.shape[1]
    idx2d = x.reshape(B // _CHUNK, _CHUNK)
    wide = _make(table.shape[0], table.shape[1], B)(idx2d, table)
    out = wide.reshape(x.shape[0], x.shape[1], _WIDE)
    return out[:, :, :table.shape[1]]


# speedup vs baseline: 1.0076x; 1.0076x over previous
import functools

import jax
import jax.numpy as jnp
from jax import lax
from jax.experimental import pallas as pl
from jax.experimental.pallas import tpu as pltpu
from jax.experimental.pallas import tpu_sc as plsc

_INFO = plsc.get_sparse_core_info()
_NC, _NS, _L = _INFO.num_cores, _INFO.num_subcores, _INFO.num_lanes
_NW = _NC * _NS

_CHUNK = 128
_NBUF = 4
_RUNROLL = 8
_WIDE = 128


@functools.lru_cache(maxsize=None)
def _make(V, D, B):
    assert B % (_NW * _CHUNK * _NBUF) == 0 and D % _L == 0
    b_per_w = B // _NW
    steps = b_per_w // _CHUNK
    rounds = steps // _NBUF
    mesh = plsc.VectorSubcoreMesh(core_axis_name="c", subcore_axis_name="s")

    scratch = (
        [pltpu.VMEM((steps, _CHUNK), jnp.int32)]
        + [pltpu.VMEM((_CHUNK, D), jnp.float32) for _ in range(2 * _NBUF)]
        + [pltpu.SemaphoreType.DMA for _ in range(2 * _NBUF)]
    )

    @functools.partial(
        pl.kernel,
        out_type=jax.ShapeDtypeStruct((B, _WIDE), jnp.float32),
        mesh=mesh,
        scratch_types=scratch,
        compiler_params=pltpu.CompilerParams(use_tc_tiling_on_sc=False),
    )
    def emb_kernel(idx_hbm, table_hbm, out_hbm, idx_v, *bufs_and_sems):
        in_bufs = bufs_and_sems[:_NBUF]
        out_bufs = bufs_and_sems[_NBUF:2 * _NBUF]
        g_sems = bufs_and_sems[2 * _NBUF:3 * _NBUF]
        w_sems = bufs_and_sems[3 * _NBUF:4 * _NBUF]

        wid = lax.axis_index("s") * _NC + lax.axis_index("c")
        row_base = wid * b_per_w

        pltpu.sync_copy(idx_hbm.at[pl.ds(wid * steps, steps)], idx_v)

        def fire_gather(b, c):
            pltpu.async_copy(table_hbm.at[idx_v.at[c]], in_bufs[b], g_sems[b])

        def wait_gather(b):
            pltpu.make_async_copy(
                table_hbm.at[pl.ds(0, _CHUNK)], in_bufs[b], g_sems[b]).wait()

        def out_slice(c):
            return out_hbm.at[pl.ds(row_base + c * _CHUNK, _CHUNK),
                              pl.ds(0, D)]

        def fire_wb(b, c):
            pltpu.async_copy(out_bufs[b], out_slice(c), w_sems[b])

        def wait_wb(b):
            pltpu.make_async_copy(out_bufs[b], out_slice(0), w_sems[b]).wait()

        def relu_chunk(b):
            src, dst = in_bufs[b], out_bufs[b]

            def body(i, carry):
                r0 = i * _RUNROLL
                for r in range(_RUNROLL):
                    for j in range(D // _L):
                        sl = pl.ds(j * _L, _L)
                        dst[r0 + r, sl] = jnp.maximum(src[r0 + r, sl], 0.0)
                return carry

            lax.fori_loop(0, _CHUNK // _RUNROLL, body, 0)

        for b in range(_NBUF):
            fire_gather(b, b)

        for b in range(_NBUF):
            wait_gather(b)
            relu_chunk(b)
            fire_wb(b, b)
            fire_gather(b, _NBUF + b)

        def round_body(p, carry):
            c0 = p * _NBUF
            for b in range(_NBUF):
                wait_gather(b)
                wait_wb(b)
                relu_chunk(b)
                fire_wb(b, c0 + b)
                fire_gather(b, c0 + _NBUF + b)
            return carry

        lax.fori_loop(1, rounds - 1, round_body, 0)

        c0 = (rounds - 1) * _NBUF
        for b in range(_NBUF):
            wait_gather(b)
            wait_wb(b)
            relu_chunk(b)
            fire_wb(b, c0 + b)
        for b in range(_NBUF):
            wait_wb(b)

    return emb_kernel


def kernel(x, table):
    B = x.shape[0] * x.shape[1]
    idx2d = x.reshape(B // _CHUNK, _CHUNK)
    wide = _make(table.shape[0], table.shape[1], B)(idx2d, table)
    out = wide.reshape(x.shape[0], x.shape[1], _WIDE)
    return out[:, :, :table.shape[1]]
